# baseline (speedup 1.0000x reference)
import functools

import jax
import jax.numpy as jnp
from jax import lax
from jax.experimental import pallas as pl
from jax.experimental.pallas import tpu as pltpu
from jax.experimental.pallas import tpu_sc as plsc

N = 10000
E = 320000
D = 128
CW = 16
NC = 2
NS = 16
NW = NC * NS
EPW = E // NW
B = 128
NCHF = EPW // B
TB = EPW - NCHF * B
N_PAD = 10240
RPT = N_PAD // NS


@functools.lru_cache(maxsize=None)
def _make_sc_aggregate(with_count):
    mesh = plsc.VectorSubcoreMesh(core_axis_name="c", subcore_axis_name="s",
                                  num_cores=NC, num_subcores=NS)

    out_type = [jax.ShapeDtypeStruct((NC, N_PAD, D), jnp.bfloat16)]
    scratch = [
        pltpu.VMEM((4, 2, B), jnp.int32),
        pltpu.VMEM((2, B, D), jnp.bfloat16),
        pltpu.VMEM((2, TB), jnp.int32),
        pltpu.VMEM((TB, D), jnp.bfloat16),
        pltpu.SemaphoreType.DMA,
        pltpu.SemaphoreType.DMA,
        pltpu.SemaphoreType.DMA,
        pltpu.SemaphoreType.DMA,
        pltpu.SemaphoreType.DMA,
        pltpu.SemaphoreType.DMA,
        pltpu.VMEM_SHARED((N_PAD, D), jnp.bfloat16),
    ]
    if with_count:
        out_type.append(jax.ShapeDtypeStruct((NC, N_PAD, CW), jnp.float32))
        scratch += [
            pltpu.VMEM((B, CW), jnp.float32),
            pltpu.VMEM_SHARED((N_PAD, CW), jnp.float32),
        ]

    @functools.partial(
        pl.kernel,
        out_type=tuple(out_type) if with_count else out_type[0],
        mesh=mesh,
        scratch_types=scratch,
        compiler_params=pltpu.CompilerParams(use_tc_tiling_on_sc=False),
    )
    def agg_kernel(x_hbm, ei_hbm, zf_hbm, *rest):
        if with_count:
            (zc_hbm, out_hbm, cnt_hbm,
             idx, rows, tidx, trows, i0, i1, i2, i3, r0, r1,
             acc, ones_v, cacc) = rest
        else:
            (out_hbm, idx, rows, tidx, trows, i0, i1, i2, i3, r0, r1,
             acc) = rest
        isem = (i0, i1, i2, i3)
        rsem = (r0, r1)
        cid = lax.axis_index("c")
        sid = lax.axis_index("s")
        wid = sid * NC + cid

        sl = pl.ds(sid * RPT, RPT)
        pltpu.sync_copy(zf_hbm, acc.at[sl])
        if with_count:
            pltpu.sync_copy(zc_hbm, cacc.at[sl])

            def fill_ones(i, _):
                ones_v[i] = jnp.ones((CW,), jnp.float32)
                return ()

            lax.fori_loop(0, B, fill_ones, ())
        plsc.subcore_barrier()

        ebase = wid * EPW

        def idxload(i, slot):
            sl_e = pl.ds(ebase + i * B, B)
            pltpu.async_copy(ei_hbm.at[0, sl_e], idx.at[slot, 0], isem[slot])
            pltpu.async_copy(ei_hbm.at[1, sl_e], idx.at[slot, 1], isem[slot])

        def wait_idx(i, slot):
            sl_e = pl.ds(ebase + i * B, B)
            pltpu.make_async_copy(ei_hbm.at[0, sl_e], idx.at[slot, 0],
                                  isem[slot]).wait()
            pltpu.make_async_copy(ei_hbm.at[1, sl_e], idx.at[slot, 1],
                                  isem[slot]).wait()

        def gather(rb, slot):
            pltpu.async_copy(x_hbm.at[idx.at[slot, 0]], rows.at[rb], rsem[rb])

        def wait_gather(rb, slot):
            pltpu.make_async_copy(x_hbm.at[idx.at[slot, 0]], rows.at[rb],
                                  rsem[rb]).wait()

        def scat(rb, slot):
            pltpu.sync_copy(rows.at[rb], acc.at[idx.at[slot, 1]], add=True)
            if with_count:
                pltpu.sync_copy(ones_v, cacc.at[idx.at[slot, 1]], add=True)

        for k in range(4):
            idxload(k, k)
        for k in range(2):
            wait_idx(k, k)
            gather(k, k)

        def body(j, _):
            base = 4 * j
            for k in range(4):
                i = base + k
                rb, slot = k % 2, k
                wait_gather(rb, slot)
                scat(rb, slot)
                idxload(i + 4, slot)
                wait_idx(i + 2, (k + 2) % 4)
                gather(rb, (k + 2) % 4)
            return ()

        lax.fori_loop(0, (NCHF - 6) // 4, body, ())

        for k in range(6):
            i = NCHF - 6 + k
            rb, slot = i % 2, i % 4
            wait_gather(rb, slot)
            scat(rb, slot)
            if k < 2:
                idxload(i + 4, slot)
            if k < 4:
                wait_idx(i + 2, (i + 2) % 4)
                gather(rb, (i + 2) % 4)

        tl = pl.ds(ebase + NCHF * B, TB)
        pltpu.async_copy(ei_hbm.at[0, tl], tidx.at[0], i0)
        pltpu.async_copy(ei_hbm.at[1, tl], tidx.at[1], i0)
        pltpu.make_async_copy(ei_hbm.at[0, tl], tidx.at[0], i0).wait()
        pltpu.make_async_copy(ei_hbm.at[1, tl], tidx.at[1], i0).wait()
        pltpu.async_copy(x_hbm.at[tidx.at[0]], trows, r0)
        pltpu.make_async_copy(x_hbm.at[tidx.at[0]], trows, r0).wait()
        pltpu.sync_copy(trows, acc.at[tidx.at[1]], add=True)
        if with_count:
            pltpu.sync_copy(ones_v.at[pl.ds(0, TB)],
                            cacc.at[tidx.at[1]], add=True)

        plsc.subcore_barrier()
        pltpu.sync_copy(acc.at[sl], out_hbm.at[cid, sl])
        if with_count:
            pltpu.sync_copy(cacc.at[sl], cnt_hbm.at[cid, sl])

    return agg_kernel


_ROWS = 2000
_GRID = N // _ROWS


def _tc1_body(p_ref, c_ref, x_ref, wl_ref, bl_ref, wr_ref, h_ref):
    p = p_ref[...].astype(jnp.float32)
    s = p[0] + p[1]
    c = c_ref[...]
    cnt = (c[0] + c[1])[:, 0:1]
    inv = 1.0 / jnp.maximum(cnt, 1.0)
    agg = s * inv
    out = (lax.dot_general(agg, wl_ref[...], (((1,), (1,)), ((), ())),
                           preferred_element_type=jnp.float32)
           + bl_ref[...]
           + lax.dot_general(x_ref[...], wr_ref[...], (((1,), (1,)), ((), ())),
                             preferred_element_type=jnp.float32))
    norm = jnp.sqrt(jnp.sum(out * out, axis=1, keepdims=True))
    out = out / jnp.maximum(norm, 1e-12)
    h_ref[...] = jnp.maximum(out, 0.0).astype(jnp.bfloat16)


def _tc2_body(q_ref, c_ref, h_ref, wl_ref, bl_ref, wr_ref, fcw_ref, fcb_ref, o_ref):
    q = q_ref[...].astype(jnp.float32)
    s2 = q[0] + q[1]
    c = c_ref[...]
    cnt = (c[0] + c[1])[:, 0:1]
    inv = 1.0 / jnp.maximum(cnt, 1.0)
    agg = s2 * inv
    out = (lax.dot_general(agg, wl_ref[...], (((1,), (1,)), ((), ())),
                           preferred_element_type=jnp.float32)
           + bl_ref[...]
           + lax.dot_general(h_ref[...].astype(jnp.float32), wr_ref[...], (((1,), (1,)), ((), ())),
                             preferred_element_type=jnp.float32))
    norm = jnp.sqrt(jnp.sum(out * out, axis=1, keepdims=True))
    out = out / jnp.maximum(norm, 1e-12)
    logits = lax.dot_general(out, fcw_ref[...], (((1,), (1,)), ((), ())),
                             preferred_element_type=jnp.float32) + fcb_ref[...]
    m = jnp.max(logits, axis=1, keepdims=True)
    e = jnp.exp(logits - m)
    o_ref[...] = e / jnp.sum(e, axis=1, keepdims=True)


def _row_blocked(width):
    return pl.BlockSpec((_ROWS, width), lambda i: (i, 0))


def _partial_blocked(width):
    return pl.BlockSpec((NC, _ROWS, width), lambda i: (0, i, 0))


def _whole(shape):
    return pl.BlockSpec(shape, lambda i: tuple(0 for _ in shape))


def kernel(feat, edge_index, W1l, b1l, W1r, W2l, b2l, W2r, fcW, fcb):
    ei = edge_index.astype(jnp.int32)

    feat_bf = feat.astype(jnp.bfloat16)
    zf = jnp.zeros((RPT, D), jnp.bfloat16)
    zc = jnp.zeros((RPT, CW), jnp.float32)

    part1, cnt1 = _make_sc_aggregate(True)(feat_bf, ei, zf, zc)

    h = pl.pallas_call(
        _tc1_body,
        grid=(_GRID,),
        in_specs=[_partial_blocked(D), _partial_blocked(CW), _row_blocked(D),
                  _whole((D, D)), _whole((1, D)), _whole((D, D))],
        out_specs=_row_blocked(D),
        out_shape=jax.ShapeDtypeStruct((N, D), jnp.bfloat16),
    )(part1, cnt1, feat, W1l, b1l.reshape(1, D), W1r)

    part2 = _make_sc_aggregate(False)(h, ei, zf)

    out = pl.pallas_call(
        _tc2_body,
        grid=(_GRID,),
        in_specs=[_partial_blocked(D), _partial_blocked(CW), _row_blocked(D),
                  _whole((D, D)), _whole((1, D)), _whole((D, D)),
                  _whole((2, D)), _whole((1, 2))],
        out_specs=_row_blocked(2),
        out_shape=jax.ShapeDtypeStruct((N, 2), jnp.float32),
    )(part2, cnt1, h, W2l, b2l.reshape(1, D), W2r, fcW, fcb.reshape(1, 2))

    return out


# speedup vs baseline: 1.1023x; 1.1023x over previous
import functools

import jax
import jax.numpy as jnp
from jax import lax
from jax.experimental import pallas as pl
from jax.experimental.pallas import tpu as pltpu
from jax.experimental.pallas import tpu_sc as plsc

N = 10000
E = 320000
D = 128
CW = 16
NC = 2
NS = 16
NW = NC * NS
EPW = E // NW
B = 128
NCHF = EPW // B
TB = EPW - NCHF * B
N_PAD = 10240
RPT = N_PAD // NS


@functools.lru_cache(maxsize=None)
def _make_sc_aggregate(with_count):
    mesh = plsc.VectorSubcoreMesh(core_axis_name="c", subcore_axis_name="s",
                                  num_cores=NC, num_subcores=NS)

    out_type = [jax.ShapeDtypeStruct((NC, N_PAD, D), jnp.bfloat16)]
    scratch = [
        pltpu.VMEM((4, 2, B), jnp.int32),
        pltpu.VMEM((4, B, D), jnp.bfloat16),
        pltpu.VMEM((2, TB), jnp.int32),
        pltpu.VMEM((TB, D), jnp.bfloat16),
        pltpu.SemaphoreType.DMA,
        pltpu.SemaphoreType.DMA,
        pltpu.SemaphoreType.DMA,
        pltpu.SemaphoreType.DMA,
        pltpu.SemaphoreType.DMA,
        pltpu.SemaphoreType.DMA,
        pltpu.SemaphoreType.DMA,
        pltpu.SemaphoreType.DMA,
        pltpu.VMEM_SHARED((N_PAD, D), jnp.bfloat16),
    ]
    if with_count:
        out_type.append(jax.ShapeDtypeStruct((NC, N_PAD, CW), jnp.float32))
        scratch += [
            pltpu.VMEM((B, CW), jnp.float32),
            pltpu.VMEM_SHARED((N_PAD, CW), jnp.float32),
        ]

    @functools.partial(
        pl.kernel,
        out_type=tuple(out_type) if with_count else out_type[0],
        mesh=mesh,
        scratch_types=scratch,
        compiler_params=pltpu.CompilerParams(use_tc_tiling_on_sc=False),
    )
    def agg_kernel(x_hbm, ei_hbm, zf_hbm, *rest):
        if with_count:
            (zc_hbm, out_hbm, cnt_hbm,
             idx, rows, tidx, trows, i0, i1, i2, i3, r0, r1, r2, r3,
             acc, ones_v, cacc) = rest
        else:
            (out_hbm, idx, rows, tidx, trows, i0, i1, i2, i3, r0, r1, r2, r3,
             acc) = rest
        isem = (i0, i1, i2, i3)
        rsem = (r0, r1, r2, r3)
        cid = lax.axis_index("c")
        sid = lax.axis_index("s")
        wid = sid * NC + cid

        sl = pl.ds(sid * RPT, RPT)
        pltpu.sync_copy(zf_hbm, acc.at[sl])
        if with_count:
            pltpu.sync_copy(zc_hbm, cacc.at[sl])

            def fill_ones(i, _):
                ones_v[i] = jnp.ones((CW,), jnp.float32)
                return ()

            lax.fori_loop(0, B, fill_ones, ())
        plsc.subcore_barrier()

        ebase = wid * EPW

        def idxload(i, slot):
            sl_e = pl.ds(ebase + i * B, B)
            pltpu.async_copy(ei_hbm.at[0, sl_e], idx.at[slot, 0], isem[slot])
            pltpu.async_copy(ei_hbm.at[1, sl_e], idx.at[slot, 1], isem[slot])

        def wait_idx(i, slot):
            sl_e = pl.ds(ebase + i * B, B)
            pltpu.make_async_copy(ei_hbm.at[0, sl_e], idx.at[slot, 0],
                                  isem[slot]).wait()
            pltpu.make_async_copy(ei_hbm.at[1, sl_e], idx.at[slot, 1],
                                  isem[slot]).wait()

        def gather(rb, slot):
            pltpu.async_copy(x_hbm.at[idx.at[slot, 0]], rows.at[rb], rsem[rb])

        def wait_gather(rb, slot):
            pltpu.make_async_copy(x_hbm.at[idx.at[slot, 0]], rows.at[rb],
                                  rsem[rb]).wait()

        def scat(rb, slot):
            pltpu.sync_copy(rows.at[rb], acc.at[idx.at[slot, 1]], add=True)
            if with_count:
                pltpu.sync_copy(ones_v, cacc.at[idx.at[slot, 1]], add=True)

        for k in range(4):
            idxload(k, k)
        for k in range(3):
            wait_idx(k, k)
            gather(k, k)

        def body(j, _):
            base = 4 * j
            for k in range(4):
                i = base + k
                slot = k
                wait_gather(slot, slot)
                scat(slot, slot)
                idxload(i + 4, slot)
                wait_idx(i + 3, (k + 3) % 4)
                gather((k + 3) % 4, (k + 3) % 4)
            return ()

        lax.fori_loop(0, (NCHF - 6) // 4, body, ())

        for k in range(6):
            i = NCHF - 6 + k
            slot = i % 4
            wait_gather(slot, slot)
            scat(slot, slot)
            if k < 2:
                idxload(i + 4, slot)
            if k < 3:
                wait_idx(i + 3, (i + 3) % 4)
                gather((i + 3) % 4, (i + 3) % 4)

        tl = pl.ds(ebase + NCHF * B, TB)
        pltpu.async_copy(ei_hbm.at[0, tl], tidx.at[0], i0)
        pltpu.async_copy(ei_hbm.at[1, tl], tidx.at[1], i0)
        pltpu.make_async_copy(ei_hbm.at[0, tl], tidx.at[0], i0).wait()
        pltpu.make_async_copy(ei_hbm.at[1, tl], tidx.at[1], i0).wait()
        pltpu.async_copy(x_hbm.at[tidx.at[0]], trows, r0)
        pltpu.make_async_copy(x_hbm.at[tidx.at[0]], trows, r0).wait()
        pltpu.sync_copy(trows, acc.at[tidx.at[1]], add=True)
        if with_count:
            pltpu.sync_copy(ones_v.at[pl.ds(0, TB)],
                            cacc.at[tidx.at[1]], add=True)

        plsc.subcore_barrier()
        pltpu.sync_copy(acc.at[sl], out_hbm.at[cid, sl])
        if with_count:
            pltpu.sync_copy(cacc.at[sl], cnt_hbm.at[cid, sl])

    return agg_kernel


_ROWS = 2000
_GRID = N // _ROWS


def _tc1_body(p_ref, c_ref, x_ref, wl_ref, bl_ref, wr_ref, h_ref):
    p = p_ref[...].astype(jnp.float32)
    s = p[0] + p[1]
    c = c_ref[...]
    cnt = (c[0] + c[1])[:, 0:1]
    inv = 1.0 / jnp.maximum(cnt, 1.0)
    agg = s * inv
    out = (lax.dot_general(agg, wl_ref[...], (((1,), (1,)), ((), ())),
                           preferred_element_type=jnp.float32)
           + bl_ref[...]
           + lax.dot_general(x_ref[...], wr_ref[...], (((1,), (1,)), ((), ())),
                             preferred_element_type=jnp.float32))
    norm = jnp.sqrt(jnp.sum(out * out, axis=1, keepdims=True))
    out = out / jnp.maximum(norm, 1e-12)
    h_ref[...] = jnp.maximum(out, 0.0).astype(jnp.bfloat16)


def _tc2_body(q_ref, c_ref, h_ref, wl_ref, bl_ref, wr_ref, fcw_ref, fcb_ref, o_ref):
    q = q_ref[...].astype(jnp.float32)
    s2 = q[0] + q[1]
    c = c_ref[...]
    cnt = (c[0] + c[1])[:, 0:1]
    inv = 1.0 / jnp.maximum(cnt, 1.0)
    agg = s2 * inv
    out = (lax.dot_general(agg, wl_ref[...], (((1,), (1,)), ((), ())),
                           preferred_element_type=jnp.float32)
           + bl_ref[...]
           + lax.dot_general(h_ref[...].astype(jnp.float32), wr_ref[...], (((1,), (1,)), ((), ())),
                             preferred_element_type=jnp.float32))
    norm = jnp.sqrt(jnp.sum(out * out, axis=1, keepdims=True))
    out = out / jnp.maximum(norm, 1e-12)
    logits = lax.dot_general(out, fcw_ref[...], (((1,), (1,)), ((), ())),
                             preferred_element_type=jnp.float32) + fcb_ref[...]
    m = jnp.max(logits, axis=1, keepdims=True)
    e = jnp.exp(logits - m)
    o_ref[...] = e / jnp.sum(e, axis=1, keepdims=True)


def _row_blocked(width):
    return pl.BlockSpec((_ROWS, width), lambda i: (i, 0))


def _partial_blocked(width):
    return pl.BlockSpec((NC, _ROWS, width), lambda i: (0, i, 0))


def _whole(shape):
    return pl.BlockSpec(shape, lambda i: tuple(0 for _ in shape))


def kernel(feat, edge_index, W1l, b1l, W1r, W2l, b2l, W2r, fcW, fcb):
    ei = edge_index.astype(jnp.int32)

    feat_bf = feat.astype(jnp.bfloat16)
    zf = jnp.zeros((RPT, D), jnp.bfloat16)
    zc = jnp.zeros((RPT, CW), jnp.float32)

    part1, cnt1 = _make_sc_aggregate(True)(feat_bf, ei, zf, zc)

    h = pl.pallas_call(
        _tc1_body,
        grid=(_GRID,),
        in_specs=[_partial_blocked(D), _partial_blocked(CW), _row_blocked(D),
                  _whole((D, D)), _whole((1, D)), _whole((D, D))],
        out_specs=_row_blocked(D),
        out_shape=jax.ShapeDtypeStruct((N, D), jnp.bfloat16),
    )(part1, cnt1, feat, W1l, b1l.reshape(1, D), W1r)

    part2 = _make_sc_aggregate(False)(h, ei, zf)

    out = pl.pallas_call(
        _tc2_body,
        grid=(_GRID,),
        in_specs=[_partial_blocked(D), _partial_blocked(CW), _row_blocked(D),
                  _whole((D, D)), _whole((1, D)), _whole((D, D)),
                  _whole((2, D)), _whole((1, 2))],
        out_specs=_row_blocked(2),
        out_shape=jax.ShapeDtypeStruct((N, 2), jnp.float32),
    )(part2, cnt1, h, W2l, b2l.reshape(1, D), W2r, fcW, fcb.reshape(1, 2))

    return out


# speedup vs baseline: 1.1104x; 1.0074x over previous
import functools

import jax
import jax.numpy as jnp
from jax import lax
from jax.experimental import pallas as pl
from jax.experimental.pallas import tpu as pltpu
from jax.experimental.pallas import tpu_sc as plsc

N = 10000
E = 320000
D = 128
CW = 16
NC = 2
NS = 16
NW = NC * NS
EPW = E // NW
B = 128
NCHF = EPW // B
TB = EPW - NCHF * B
N_PAD = 10240
RPT = N_PAD // NS


@functools.lru_cache(maxsize=None)
def _make_sc_aggregate(with_count):
    mesh = plsc.VectorSubcoreMesh(core_axis_name="c", subcore_axis_name="s",
                                  num_cores=NC, num_subcores=NS)

    out_type = [jax.ShapeDtypeStruct((NC, N_PAD, D), jnp.bfloat16)]
    scratch = [
        pltpu.VMEM((4, 2, B), jnp.int32),
        pltpu.VMEM((4, B, D), jnp.bfloat16),
        pltpu.VMEM((2, TB), jnp.int32),
        pltpu.VMEM((TB, D), jnp.bfloat16),
        pltpu.SemaphoreType.DMA,
        pltpu.SemaphoreType.DMA,
        pltpu.SemaphoreType.DMA,
        pltpu.SemaphoreType.DMA,
        pltpu.SemaphoreType.DMA,
        pltpu.SemaphoreType.DMA,
        pltpu.SemaphoreType.DMA,
        pltpu.SemaphoreType.DMA,
        pltpu.VMEM_SHARED((N_PAD, D), jnp.bfloat16),
    ]
    if with_count:
        out_type.append(jax.ShapeDtypeStruct((NC, N_PAD, CW), jnp.float32))
        scratch += [
            pltpu.VMEM((B, CW), jnp.float32),
            pltpu.VMEM_SHARED((N_PAD, CW), jnp.float32),
        ]

    @functools.partial(
        pl.kernel,
        out_type=tuple(out_type) if with_count else out_type[0],
        mesh=mesh,
        scratch_types=scratch,
        compiler_params=pltpu.CompilerParams(use_tc_tiling_on_sc=False),
    )
    def agg_kernel(x_hbm, ei_hbm, zf_hbm, *rest):
        if with_count:
            (zc_hbm, out_hbm, cnt_hbm,
             idx, rows, tidx, trows, i0, i1, i2, i3, r0, r1, r2, r3,
             acc, ones_v, cacc) = rest
        else:
            (out_hbm, idx, rows, tidx, trows, i0, i1, i2, i3, r0, r1, r2, r3,
             acc) = rest
        isem = (i0, i1, i2, i3)
        rsem = (r0, r1, r2, r3)
        cid = lax.axis_index("c")
        sid = lax.axis_index("s")
        wid = sid * NC + cid

        sl = pl.ds(sid * RPT, RPT)
        pltpu.sync_copy(zf_hbm, acc.at[sl])
        if with_count:
            pltpu.sync_copy(zc_hbm, cacc.at[sl])

            def fill_ones(i, _):
                ones_v[i] = jnp.ones((CW,), jnp.float32)
                return ()

            lax.fori_loop(0, B, fill_ones, ())
        plsc.subcore_barrier()

        ebase = wid * EPW

        def idxload(i, slot):
            sl_e = pl.ds(ebase + i * B, B)
            pltpu.async_copy(ei_hbm.at[0, sl_e], idx.at[slot, 0], isem[slot])
            pltpu.async_copy(ei_hbm.at[1, sl_e], idx.at[slot, 1], isem[slot])

        def wait_idx(i, slot):
            sl_e = pl.ds(ebase + i * B, B)
            pltpu.make_async_copy(ei_hbm.at[0, sl_e], idx.at[slot, 0],
                                  isem[slot]).wait()
            pltpu.make_async_copy(ei_hbm.at[1, sl_e], idx.at[slot, 1],
                                  isem[slot]).wait()

        def gather(rb, slot):
            pltpu.async_copy(x_hbm.at[idx.at[slot, 0]], rows.at[rb], rsem[rb])

        def wait_gather(rb, slot):
            pltpu.make_async_copy(x_hbm.at[idx.at[slot, 0]], rows.at[rb],
                                  rsem[rb]).wait()

        def scat(rb, slot):
            pltpu.sync_copy(rows.at[rb], acc.at[idx.at[slot, 1]], add=True)
            if with_count:
                pltpu.sync_copy(ones_v, cacc.at[idx.at[slot, 1]], add=True)

        for k in range(4):
            idxload(k, k)
        for k in range(3):
            wait_idx(k, k)
            gather(k, k)

        def body(j, _):
            base = 4 * j
            for k in range(4):
                i = base + k
                slot = k
                wait_gather(slot, slot)
                scat(slot, slot)
                idxload(i + 4, slot)
                wait_idx(i + 3, (k + 3) % 4)
                gather((k + 3) % 4, (k + 3) % 4)
            return ()

        lax.fori_loop(0, (NCHF - 6) // 4, body, ())

        for k in range(6):
            i = NCHF - 6 + k
            slot = i % 4
            wait_gather(slot, slot)
            scat(slot, slot)
            if k < 2:
                idxload(i + 4, slot)
            if k < 3:
                wait_idx(i + 3, (i + 3) % 4)
                gather((i + 3) % 4, (i + 3) % 4)

        tl = pl.ds(ebase + NCHF * B, TB)
        pltpu.async_copy(ei_hbm.at[0, tl], tidx.at[0], i0)
        pltpu.async_copy(ei_hbm.at[1, tl], tidx.at[1], i0)
        pltpu.make_async_copy(ei_hbm.at[0, tl], tidx.at[0], i0).wait()
        pltpu.make_async_copy(ei_hbm.at[1, tl], tidx.at[1], i0).wait()
        pltpu.async_copy(x_hbm.at[tidx.at[0]], trows, r0)
        pltpu.make_async_copy(x_hbm.at[tidx.at[0]], trows, r0).wait()
        pltpu.sync_copy(trows, acc.at[tidx.at[1]], add=True)
        if with_count:
            pltpu.sync_copy(ones_v.at[pl.ds(0, TB)],
                            cacc.at[tidx.at[1]], add=True)

        plsc.subcore_barrier()
        pltpu.sync_copy(acc.at[sl], out_hbm.at[cid, sl])
        if with_count:
            pltpu.sync_copy(cacc.at[sl], cnt_hbm.at[cid, sl])

    return agg_kernel


_ROWS = 5000
_GRID = N // _ROWS


def _tc1_body(p_ref, c_ref, x_ref, wl_ref, bl_ref, wr_ref, h_ref):
    p = p_ref[...].astype(jnp.float32)
    s = p[0] + p[1]
    c = c_ref[...]
    cnt = (c[0] + c[1])[:, 0:1]
    inv = 1.0 / jnp.maximum(cnt, 1.0)
    agg = s * inv
    out = (lax.dot_general(agg, wl_ref[...], (((1,), (1,)), ((), ())),
                           preferred_element_type=jnp.float32)
           + bl_ref[...]
           + lax.dot_general(x_ref[...], wr_ref[...], (((1,), (1,)), ((), ())),
                             preferred_element_type=jnp.float32))
    norm = jnp.sqrt(jnp.sum(out * out, axis=1, keepdims=True))
    out = out / jnp.maximum(norm, 1e-12)
    h_ref[...] = jnp.maximum(out, 0.0).astype(jnp.bfloat16)


def _tc2_body(q_ref, c_ref, h_ref, wl_ref, bl_ref, wr_ref, fcw_ref, fcb_ref, o_ref):
    q = q_ref[...].astype(jnp.float32)
    s2 = q[0] + q[1]
    c = c_ref[...]
    cnt = (c[0] + c[1])[:, 0:1]
    inv = 1.0 / jnp.maximum(cnt, 1.0)
    agg = s2 * inv
    out = (lax.dot_general(agg, wl_ref[...], (((1,), (1,)), ((), ())),
                           preferred_element_type=jnp.float32)
           + bl_ref[...]
           + lax.dot_general(h_ref[...].astype(jnp.float32), wr_ref[...], (((1,), (1,)), ((), ())),
                             preferred_element_type=jnp.float32))
    norm = jnp.sqrt(jnp.sum(out * out, axis=1, keepdims=True))
    out = out / jnp.maximum(norm, 1e-12)
    logits = lax.dot_general(out, fcw_ref[...], (((1,), (1,)), ((), ())),
                             preferred_element_type=jnp.float32) + fcb_ref[...]
    m = jnp.max(logits, axis=1, keepdims=True)
    e = jnp.exp(logits - m)
    o_ref[...] = e / jnp.sum(e, axis=1, keepdims=True)


def _row_blocked(width):
    return pl.BlockSpec((_ROWS, width), lambda i: (i, 0))


def _partial_blocked(width):
    return pl.BlockSpec((NC, _ROWS, width), lambda i: (0, i, 0))


def _whole(shape):
    return pl.BlockSpec(shape, lambda i: tuple(0 for _ in shape))


def kernel(feat, edge_index, W1l, b1l, W1r, W2l, b2l, W2r, fcW, fcb):
    ei = edge_index.astype(jnp.int32)

    feat_bf = feat.astype(jnp.bfloat16)
    zf = jnp.zeros((RPT, D), jnp.bfloat16)
    zc = jnp.zeros((RPT, CW), jnp.float32)

    part1, cnt1 = _make_sc_aggregate(True)(feat_bf, ei, zf, zc)

    h = pl.pallas_call(
        _tc1_body,
        grid=(_GRID,),
        in_specs=[_partial_blocked(D), _partial_blocked(CW), _row_blocked(D),
                  _whole((D, D)), _whole((1, D)), _whole((D, D))],
        out_specs=_row_blocked(D),
        out_shape=jax.ShapeDtypeStruct((N, D), jnp.bfloat16),
    )(part1, cnt1, feat, W1l, b1l.reshape(1, D), W1r)

    part2 = _make_sc_aggregate(False)(h, ei, zf)

    out = pl.pallas_call(
        _tc2_body,
        grid=(_GRID,),
        in_specs=[_partial_blocked(D), _partial_blocked(CW), _row_blocked(D),
                  _whole((D, D)), _whole((1, D)), _whole((D, D)),
                  _whole((2, D)), _whole((1, 2))],
        out_specs=_row_blocked(2),
        out_shape=jax.ShapeDtypeStruct((N, 2), jnp.float32),
    )(part2, cnt1, h, W2l, b2l.reshape(1, D), W2r, fcW, fcb.reshape(1, 2))

    return out
